# baseline (speedup 1.0000x reference)
import functools

import jax
import jax.numpy as jnp
from jax import lax
from jax.experimental import pallas as pl
from jax.experimental.pallas import tpu as pltpu
from jax.experimental.pallas import tpu_sc as plsc

A = 4096
B = 4096
NC = 2
NS = 16
NW = NC * NS
LANES = 16


def _body(n, a_hbm, b_hbm, v_hbm, tab_hbm, out_hbm,
          a_v, b_v, f_v, val_v, g_v,
          ta_v, tb_v, tf_v, tval_v, tg_v, acc_v, sem):
    chunk = (n // (NW * 8)) * 8
    rem = n - NW * chunk
    vsteps = -(-chunk // LANES)
    main_valid = chunk - (vsteps - 1) * LANES
    tsteps = -(-rem // LANES)
    tail_valid = rem - (tsteps - 1) * LANES

    cid = lax.axis_index("c")
    sid = lax.axis_index("s")
    wid = sid * NC + cid
    base = wid * chunk

    cp_a = pltpu.async_copy(a_hbm.at[pl.ds(base, chunk)],
                            a_v.at[pl.ds(0, chunk)], sem)
    cp_b = pltpu.async_copy(b_hbm.at[pl.ds(base, chunk)],
                            b_v.at[pl.ds(0, chunk)], sem)
    cp_v = pltpu.async_copy(v_hbm.at[pl.ds(base, chunk)],
                            val_v.at[pl.ds(0, chunk)], sem)
    cp_a.wait()
    cp_b.wait()
    cp_v.wait()

    lanes = lax.iota(jnp.int32, LANES)

    def idx_body(i, carry):
        s = pl.ds(i * LANES, LANES)
        f_v[s] = a_v[s] * B + b_v[s]
        return carry

    lax.fori_loop(0, vsteps, idx_body, 0)

    s_last = pl.ds((vsteps - 1) * LANES, LANES)
    keep = lanes < main_valid
    f_v[s_last] = jnp.where(keep, f_v[s_last], 0)
    val_v[s_last] = jnp.where(keep, val_v[s_last], 0.0)

    pltpu.async_copy(tab_hbm.at[f_v], g_v, sem).wait()

    def dot_body(i, acc):
        s = pl.ds(i * LANES, LANES)
        return acc + g_v[s] * val_v[s]

    acc = lax.fori_loop(0, vsteps, dot_body,
                        jnp.zeros((LANES,), jnp.float32))
    acc_v[...] = acc

    if rem:
        @pl.when(wid == NW - 1)
        def _tail():
            tbase = NW * chunk
            tp_a = pltpu.async_copy(a_hbm.at[pl.ds(tbase, rem)],
                                    ta_v.at[pl.ds(0, rem)], sem)
            tp_b = pltpu.async_copy(b_hbm.at[pl.ds(tbase, rem)],
                                    tb_v.at[pl.ds(0, rem)], sem)
            tp_v = pltpu.async_copy(v_hbm.at[pl.ds(tbase, rem)],
                                    tval_v.at[pl.ds(0, rem)], sem)
            tp_a.wait()
            tp_b.wait()
            tp_v.wait()

            def tidx_body(i, carry):
                s = pl.ds(i * LANES, LANES)
                tf_v[s] = ta_v[s] * B + tb_v[s]
                return carry

            lax.fori_loop(0, tsteps, tidx_body, 0)
            ts_last = pl.ds((tsteps - 1) * LANES, LANES)
            tkeep = lanes < tail_valid
            tf_v[ts_last] = jnp.where(tkeep, tf_v[ts_last], 0)
            tval_v[ts_last] = jnp.where(tkeep, tval_v[ts_last], 0.0)

            pltpu.async_copy(tab_hbm.at[tf_v], tg_v, sem).wait()

            def tdot_body(i, acc):
                s = pl.ds(i * LANES, LANES)
                return acc + tg_v[s] * tval_v[s]

            tacc = lax.fori_loop(0, tsteps, tdot_body,
                                 jnp.zeros((LANES,), jnp.float32))
            acc_v[...] = acc_v[...] + tacc

    pltpu.sync_copy(acc_v, out_hbm.at[pl.ds(wid * LANES, LANES)])


def kernel(a_idx, b_idx, values, coefficients):
    n = a_idx.shape[0]
    chunk = (n // (NW * 8)) * 8
    rem = n - NW * chunk
    cbuf = -(-chunk // LANES) * LANES
    tbuf = max(-(-rem // LANES) * LANES, LANES)
    tab = coefficients.reshape(A * B)

    mesh = plsc.VectorSubcoreMesh(core_axis_name="c", subcore_axis_name="s")
    out = pl.kernel(
        functools.partial(_body, n),
        out_type=jax.ShapeDtypeStruct((NW * LANES,), jnp.float32),
        mesh=mesh,
        scratch_types=[
            pltpu.VMEM((cbuf,), jnp.int32),
            pltpu.VMEM((cbuf,), jnp.int32),
            pltpu.VMEM((cbuf,), jnp.int32),
            pltpu.VMEM((cbuf,), jnp.float32),
            pltpu.VMEM((cbuf,), jnp.float32),
            pltpu.VMEM((tbuf,), jnp.int32),
            pltpu.VMEM((tbuf,), jnp.int32),
            pltpu.VMEM((tbuf,), jnp.int32),
            pltpu.VMEM((tbuf,), jnp.float32),
            pltpu.VMEM((tbuf,), jnp.float32),
            pltpu.VMEM((LANES,), jnp.float32),
            pltpu.SemaphoreType.DMA,
        ],
    )(a_idx, b_idx, values, tab)
    return jnp.sum(out)


# speedup vs baseline: 1.0113x; 1.0113x over previous
import functools

import jax
import jax.numpy as jnp
from jax import lax
from jax.experimental import pallas as pl
from jax.experimental.pallas import tpu as pltpu
from jax.experimental.pallas import tpu_sc as plsc

A = 4096
B = 4096
NC = 2
NS = 16
NW = NC * NS
LANES = 16


def _body(n, a_hbm, b_hbm, v_hbm, tab_hbm, out_hbm,
          a_v, b_v, f_v, val_v, g_v,
          ta_v, tb_v, tf_v, tval_v, tg_v, acc_v, sem):
    chunk = (n // (NW * 8)) * 8
    rem = n - NW * chunk
    vsteps = -(-chunk // LANES)
    main_valid = chunk - (vsteps - 1) * LANES
    tsteps = -(-rem // LANES)
    tail_valid = rem - (tsteps - 1) * LANES

    cid = lax.axis_index("c")
    sid = lax.axis_index("s")
    wid = sid * NC + cid
    base = wid * chunk

    cp_a = pltpu.async_copy(a_hbm.at[pl.ds(base, chunk)],
                            a_v.at[pl.ds(0, chunk)], sem)
    cp_b = pltpu.async_copy(b_hbm.at[pl.ds(base, chunk)],
                            b_v.at[pl.ds(0, chunk)], sem)
    cp_v = pltpu.async_copy(v_hbm.at[pl.ds(base, chunk)],
                            val_v.at[pl.ds(0, chunk)], sem)
    cp_a.wait()
    cp_b.wait()
    cp_v.wait()

    lanes = lax.iota(jnp.int32, LANES)

    @plsc.parallel_loop(0, vsteps * LANES, step=LANES, unroll=8)
    def idx_body(i):
        s = pl.ds(i, LANES)
        f_v[s] = a_v[s] * B + b_v[s]

    s_last = pl.ds((vsteps - 1) * LANES, LANES)
    keep = lanes < main_valid
    f_v[s_last] = jnp.where(keep, f_v[s_last], 0)
    val_v[s_last] = jnp.where(keep, val_v[s_last], 0.0)

    pltpu.async_copy(tab_hbm.at[f_v], g_v, sem).wait()

    @plsc.parallel_loop(0, vsteps * LANES, step=LANES, unroll=8,
                        carry=jnp.zeros((LANES,), jnp.float32))
    def dot_body(i, acc):
        s = pl.ds(i, LANES)
        return acc + g_v[s] * val_v[s]

    acc_v[...] = dot_body

    if rem:
        @pl.when(wid == NW - 1)
        def _tail():
            tbase = NW * chunk
            tp_a = pltpu.async_copy(a_hbm.at[pl.ds(tbase, rem)],
                                    ta_v.at[pl.ds(0, rem)], sem)
            tp_b = pltpu.async_copy(b_hbm.at[pl.ds(tbase, rem)],
                                    tb_v.at[pl.ds(0, rem)], sem)
            tp_v = pltpu.async_copy(v_hbm.at[pl.ds(tbase, rem)],
                                    tval_v.at[pl.ds(0, rem)], sem)
            tp_a.wait()
            tp_b.wait()
            tp_v.wait()

            def tidx_body(i, carry):
                s = pl.ds(i * LANES, LANES)
                tf_v[s] = ta_v[s] * B + tb_v[s]
                return carry

            lax.fori_loop(0, tsteps, tidx_body, 0)
            ts_last = pl.ds((tsteps - 1) * LANES, LANES)
            tkeep = lanes < tail_valid
            tf_v[ts_last] = jnp.where(tkeep, tf_v[ts_last], 0)
            tval_v[ts_last] = jnp.where(tkeep, tval_v[ts_last], 0.0)

            pltpu.async_copy(tab_hbm.at[tf_v], tg_v, sem).wait()

            def tdot_body(i, acc):
                s = pl.ds(i * LANES, LANES)
                return acc + tg_v[s] * tval_v[s]

            tacc = lax.fori_loop(0, tsteps, tdot_body,
                                 jnp.zeros((LANES,), jnp.float32))
            acc_v[...] = acc_v[...] + tacc

    pltpu.sync_copy(acc_v, out_hbm.at[pl.ds(wid * LANES, LANES)])


def kernel(a_idx, b_idx, values, coefficients):
    n = a_idx.shape[0]
    chunk = (n // (NW * 8)) * 8
    rem = n - NW * chunk
    cbuf = -(-chunk // LANES) * LANES
    tbuf = max(-(-rem // LANES) * LANES, LANES)
    tab = coefficients.reshape(A * B)

    mesh = plsc.VectorSubcoreMesh(core_axis_name="c", subcore_axis_name="s")
    out = pl.kernel(
        functools.partial(_body, n),
        out_type=jax.ShapeDtypeStruct((NW * LANES,), jnp.float32),
        mesh=mesh,
        scratch_types=[
            pltpu.VMEM((cbuf,), jnp.int32),
            pltpu.VMEM((cbuf,), jnp.int32),
            pltpu.VMEM((cbuf,), jnp.int32),
            pltpu.VMEM((cbuf,), jnp.float32),
            pltpu.VMEM((cbuf,), jnp.float32),
            pltpu.VMEM((tbuf,), jnp.int32),
            pltpu.VMEM((tbuf,), jnp.int32),
            pltpu.VMEM((tbuf,), jnp.int32),
            pltpu.VMEM((tbuf,), jnp.float32),
            pltpu.VMEM((tbuf,), jnp.float32),
            pltpu.VMEM((LANES,), jnp.float32),
            pltpu.SemaphoreType.DMA,
        ],
    )(a_idx, b_idx, values, tab)
    return jnp.sum(out)


# speedup vs baseline: 1.0256x; 1.0142x over previous
import functools

import jax
import jax.numpy as jnp
from jax import lax
from jax.experimental import pallas as pl
from jax.experimental.pallas import tpu as pltpu
from jax.experimental.pallas import tpu_sc as plsc

A = 4096
B = 4096
NC = 2
NS = 16
NW = NC * NS
LANES = 16


def _body(n, a_hbm, b_hbm, v_hbm, tab_hbm, out_hbm,
          a_v, b_v, f_v, val_v, g_v,
          ta_v, tb_v, tf_v, tval_v, tg_v, acc_v, sem, gsem1, gsem2):
    chunk = (n // (NW * 8)) * 8
    rem = n - NW * chunk
    vsteps = -(-chunk // LANES)
    main_valid = chunk - (vsteps - 1) * LANES
    tsteps = -(-rem // LANES)
    tail_valid = rem - (tsteps - 1) * LANES

    cid = lax.axis_index("c")
    sid = lax.axis_index("s")
    wid = sid * NC + cid
    base = wid * chunk

    cp_a = pltpu.async_copy(a_hbm.at[pl.ds(base, chunk)],
                            a_v.at[pl.ds(0, chunk)], sem)
    cp_b = pltpu.async_copy(b_hbm.at[pl.ds(base, chunk)],
                            b_v.at[pl.ds(0, chunk)], sem)
    cp_v = pltpu.async_copy(v_hbm.at[pl.ds(base, chunk)],
                            val_v.at[pl.ds(0, chunk)], sem)
    cp_a.wait()
    cp_b.wait()

    lanes = lax.iota(jnp.int32, LANES)

    h1 = (vsteps // 2) * LANES
    h2 = vsteps * LANES - h1

    @plsc.parallel_loop(0, h1, step=LANES, unroll=8)
    def idx1_body(i):
        s = pl.ds(i, LANES)
        f_v[s] = a_v[s] * B + b_v[s]

    gcp1 = pltpu.async_copy(tab_hbm.at[f_v.at[pl.ds(0, h1)]],
                            g_v.at[pl.ds(0, h1)], gsem1)

    @plsc.parallel_loop(h1, h1 + h2, step=LANES, unroll=8)
    def idx2_body(i):
        s = pl.ds(i, LANES)
        f_v[s] = a_v[s] * B + b_v[s]

    s_last = pl.ds((vsteps - 1) * LANES, LANES)
    keep = lanes < main_valid
    f_v[s_last] = jnp.where(keep, f_v[s_last], 0)
    cp_v.wait()
    val_v[s_last] = jnp.where(keep, val_v[s_last], 0.0)

    gcp2 = pltpu.async_copy(tab_hbm.at[f_v.at[pl.ds(h1, h2)]],
                            g_v.at[pl.ds(h1, h2)], gsem2)
    gcp1.wait()

    @plsc.parallel_loop(0, h1, step=LANES, unroll=8,
                        carry=jnp.zeros((LANES,), jnp.float32))
    def dot1_body(i, acc):
        s = pl.ds(i, LANES)
        return acc + g_v[s] * val_v[s]

    gcp2.wait()

    @plsc.parallel_loop(h1, h1 + h2, step=LANES, unroll=8,
                        carry=dot1_body)
    def dot2_body(i, acc):
        s = pl.ds(i, LANES)
        return acc + g_v[s] * val_v[s]

    acc_v[...] = dot2_body

    if rem:
        @pl.when(wid == NW - 1)
        def _tail():
            tbase = NW * chunk
            tp_a = pltpu.async_copy(a_hbm.at[pl.ds(tbase, rem)],
                                    ta_v.at[pl.ds(0, rem)], sem)
            tp_b = pltpu.async_copy(b_hbm.at[pl.ds(tbase, rem)],
                                    tb_v.at[pl.ds(0, rem)], sem)
            tp_v = pltpu.async_copy(v_hbm.at[pl.ds(tbase, rem)],
                                    tval_v.at[pl.ds(0, rem)], sem)
            tp_a.wait()
            tp_b.wait()
            tp_v.wait()

            def tidx_body(i, carry):
                s = pl.ds(i * LANES, LANES)
                tf_v[s] = ta_v[s] * B + tb_v[s]
                return carry

            lax.fori_loop(0, tsteps, tidx_body, 0)
            ts_last = pl.ds((tsteps - 1) * LANES, LANES)
            tkeep = lanes < tail_valid
            tf_v[ts_last] = jnp.where(tkeep, tf_v[ts_last], 0)
            tval_v[ts_last] = jnp.where(tkeep, tval_v[ts_last], 0.0)

            pltpu.async_copy(tab_hbm.at[tf_v], tg_v, sem).wait()

            def tdot_body(i, acc):
                s = pl.ds(i * LANES, LANES)
                return acc + tg_v[s] * tval_v[s]

            tacc = lax.fori_loop(0, tsteps, tdot_body,
                                 jnp.zeros((LANES,), jnp.float32))
            acc_v[...] = acc_v[...] + tacc

    pltpu.sync_copy(acc_v, out_hbm.at[pl.ds(wid * LANES, LANES)])


def kernel(a_idx, b_idx, values, coefficients):
    n = a_idx.shape[0]
    chunk = (n // (NW * 8)) * 8
    rem = n - NW * chunk
    cbuf = -(-chunk // LANES) * LANES
    tbuf = max(-(-rem // LANES) * LANES, LANES)
    tab = coefficients.reshape(A * B)

    mesh = plsc.VectorSubcoreMesh(core_axis_name="c", subcore_axis_name="s")
    out = pl.kernel(
        functools.partial(_body, n),
        out_type=jax.ShapeDtypeStruct((NW * LANES,), jnp.float32),
        mesh=mesh,
        scratch_types=[
            pltpu.VMEM((cbuf,), jnp.int32),
            pltpu.VMEM((cbuf,), jnp.int32),
            pltpu.VMEM((cbuf,), jnp.int32),
            pltpu.VMEM((cbuf,), jnp.float32),
            pltpu.VMEM((cbuf,), jnp.float32),
            pltpu.VMEM((tbuf,), jnp.int32),
            pltpu.VMEM((tbuf,), jnp.int32),
            pltpu.VMEM((tbuf,), jnp.int32),
            pltpu.VMEM((tbuf,), jnp.float32),
            pltpu.VMEM((tbuf,), jnp.float32),
            pltpu.VMEM((LANES,), jnp.float32),
            pltpu.SemaphoreType.DMA,
            pltpu.SemaphoreType.DMA,
            pltpu.SemaphoreType.DMA,
        ],
    )(a_idx, b_idx, values, tab)
    return jnp.sum(out)


# speedup vs baseline: 1.0297x; 1.0041x over previous
import functools

import jax
import jax.numpy as jnp
from jax import lax
from jax.experimental import pallas as pl
from jax.experimental.pallas import tpu as pltpu
from jax.experimental.pallas import tpu_sc as plsc

A = 4096
B = 4096
NC = 2
NS = 16
NW = NC * NS
LANES = 16


def _body(n, a_hbm, b_hbm, v_hbm, tab_hbm, out_hbm,
          a_v, b_v, f_v, val_v, g_v,
          ta_v, tb_v, tf_v, tval_v, tg_v, acc_v, sem,
          gsem1, gsem2, gsem3, gsem4):
    chunk = (n // (NW * 8)) * 8
    rem = n - NW * chunk
    vsteps = -(-chunk // LANES)
    main_valid = chunk - (vsteps - 1) * LANES
    tsteps = -(-rem // LANES)
    tail_valid = rem - (tsteps - 1) * LANES

    cid = lax.axis_index("c")
    sid = lax.axis_index("s")
    wid = sid * NC + cid
    base = wid * chunk

    cp_a = pltpu.async_copy(a_hbm.at[pl.ds(base, chunk)],
                            a_v.at[pl.ds(0, chunk)], sem)
    cp_b = pltpu.async_copy(b_hbm.at[pl.ds(base, chunk)],
                            b_v.at[pl.ds(0, chunk)], sem)
    cp_v = pltpu.async_copy(v_hbm.at[pl.ds(base, chunk)],
                            val_v.at[pl.ds(0, chunk)], sem)
    cp_a.wait()
    cp_b.wait()

    lanes = lax.iota(jnp.int32, LANES)

    nph = 4
    per = (vsteps // nph) * LANES
    bounds = [p * per for p in range(nph)] + [vsteps * LANES]
    gsems = [gsem1, gsem2, gsem3, gsem4]

    def run_idx(lo, hi, mask_tail):
        @plsc.parallel_loop(lo, hi, step=LANES, unroll=8)
        def _(i):
            s = pl.ds(i, LANES)
            f_v[s] = a_v[s] * B + b_v[s]
        if mask_tail:
            s_last = pl.ds(hi - LANES, LANES)
            keep = lanes < main_valid
            f_v[s_last] = jnp.where(keep, f_v[s_last], 0)
            val_v[s_last] = jnp.where(keep, val_v[s_last], 0.0)

    def fire_gather(p):
        lo, hi = bounds[p], bounds[p + 1]
        return pltpu.async_copy(tab_hbm.at[f_v.at[pl.ds(lo, hi - lo)]],
                                g_v.at[pl.ds(lo, hi - lo)], gsems[p])

    def run_mac(lo, hi, acc):
        @plsc.parallel_loop(lo, hi, step=LANES, unroll=8, carry=acc)
        def macs(i, a):
            s = pl.ds(i, LANES)
            return a + g_v[s] * val_v[s]
        return macs

    run_idx(bounds[0], bounds[1], False)
    gcps = [fire_gather(0)]
    run_idx(bounds[1], bounds[2], False)
    gcps.append(fire_gather(1))
    run_idx(bounds[2], bounds[3], False)
    cp_v.wait()
    run_idx(bounds[3], bounds[4], True)
    gcps.append(fire_gather(2))
    gcps.append(fire_gather(3))

    acc = jnp.zeros((LANES,), jnp.float32)
    for p in range(nph):
        gcps[p].wait()
        acc = run_mac(bounds[p], bounds[p + 1], acc)
    acc_v[...] = acc

    if rem:
        @pl.when(wid == NW - 1)
        def _tail():
            tbase = NW * chunk
            tp_a = pltpu.async_copy(a_hbm.at[pl.ds(tbase, rem)],
                                    ta_v.at[pl.ds(0, rem)], sem)
            tp_b = pltpu.async_copy(b_hbm.at[pl.ds(tbase, rem)],
                                    tb_v.at[pl.ds(0, rem)], sem)
            tp_v = pltpu.async_copy(v_hbm.at[pl.ds(tbase, rem)],
                                    tval_v.at[pl.ds(0, rem)], sem)
            tp_a.wait()
            tp_b.wait()
            tp_v.wait()

            def tidx_body(i, carry):
                s = pl.ds(i * LANES, LANES)
                tf_v[s] = ta_v[s] * B + tb_v[s]
                return carry

            lax.fori_loop(0, tsteps, tidx_body, 0)
            ts_last = pl.ds((tsteps - 1) * LANES, LANES)
            tkeep = lanes < tail_valid
            tf_v[ts_last] = jnp.where(tkeep, tf_v[ts_last], 0)
            tval_v[ts_last] = jnp.where(tkeep, tval_v[ts_last], 0.0)

            pltpu.async_copy(tab_hbm.at[tf_v], tg_v, sem).wait()

            def tdot_body(i, acc):
                s = pl.ds(i * LANES, LANES)
                return acc + tg_v[s] * tval_v[s]

            tacc = lax.fori_loop(0, tsteps, tdot_body,
                                 jnp.zeros((LANES,), jnp.float32))
            acc_v[...] = acc_v[...] + tacc

    pltpu.sync_copy(acc_v, out_hbm.at[pl.ds(wid * LANES, LANES)])


def kernel(a_idx, b_idx, values, coefficients):
    n = a_idx.shape[0]
    chunk = (n // (NW * 8)) * 8
    rem = n - NW * chunk
    cbuf = -(-chunk // LANES) * LANES
    tbuf = max(-(-rem // LANES) * LANES, LANES)
    tab = coefficients.reshape(A * B)

    mesh = plsc.VectorSubcoreMesh(core_axis_name="c", subcore_axis_name="s")
    out = pl.kernel(
        functools.partial(_body, n),
        out_type=jax.ShapeDtypeStruct((NW * LANES,), jnp.float32),
        mesh=mesh,
        scratch_types=[
            pltpu.VMEM((cbuf,), jnp.int32),
            pltpu.VMEM((cbuf,), jnp.int32),
            pltpu.VMEM((cbuf,), jnp.int32),
            pltpu.VMEM((cbuf,), jnp.float32),
            pltpu.VMEM((cbuf,), jnp.float32),
            pltpu.VMEM((tbuf,), jnp.int32),
            pltpu.VMEM((tbuf,), jnp.int32),
            pltpu.VMEM((tbuf,), jnp.int32),
            pltpu.VMEM((tbuf,), jnp.float32),
            pltpu.VMEM((tbuf,), jnp.float32),
            pltpu.VMEM((LANES,), jnp.float32),
            pltpu.SemaphoreType.DMA,
            pltpu.SemaphoreType.DMA,
            pltpu.SemaphoreType.DMA,
            pltpu.SemaphoreType.DMA,
            pltpu.SemaphoreType.DMA,
        ],
    )(a_idx, b_idx, values, tab)
    return jnp.sum(out)


# speedup vs baseline: 2.7203x; 2.6417x over previous
import functools

import jax
import jax.numpy as jnp
from jax import lax
from jax.experimental import pallas as pl
from jax.experimental.pallas import tpu as pltpu
from jax.experimental.pallas import tpu_sc as plsc

A = 4096
B = 4096
NC = 2
NS = 16
NW = NC * NS
LANES = 16


def _body(n, a_hbm, b_hbm, v_hbm, tab_hbm, out_hbm,
          a_v, b_v, f_v, val_v, g_v,
          ta_v, tb_v, tf_v, tval_v, tg_v, acc_v, sem,
          gsem1, gsem2, gsem3, gsem4):
    chunk = (n // (NW * 8)) * 8
    rem = n - NW * chunk
    vsteps = -(-chunk // LANES)
    main_valid = chunk - (vsteps - 1) * LANES
    tsteps = -(-rem // LANES)
    tail_valid = rem - (tsteps - 1) * LANES

    cid = lax.axis_index("c")
    sid = lax.axis_index("s")
    wid = sid * NC + cid
    base = wid * chunk

    cp_a = pltpu.async_copy(a_hbm.at[pl.ds(base, chunk)],
                            a_v.at[pl.ds(0, chunk)], sem)
    cp_b = pltpu.async_copy(b_hbm.at[pl.ds(base, chunk)],
                            b_v.at[pl.ds(0, chunk)], sem)
    cp_v = pltpu.async_copy(v_hbm.at[pl.ds(base, chunk)],
                            val_v.at[pl.ds(0, chunk)], sem)
    cp_a.wait()
    cp_b.wait()

    lanes = lax.iota(jnp.int32, LANES)

    nph = 4
    per = (vsteps // nph) * LANES
    bounds = [p * per for p in range(nph)] + [vsteps * LANES]
    gsems = [gsem1, gsem2, gsem3, gsem4]

    def run_idx(lo, hi, mask_tail):
        @plsc.parallel_loop(lo, hi, step=LANES, unroll=8)
        def _(i):
            s = pl.ds(i, LANES)
            a = a_v[s]
            b = b_v[s]
            f_v[s] = (((a >> 3) << 15) + ((b >> 7) << 10)
                      + ((a & 7) << 7) + (b & 127))
        if mask_tail:
            s_last = pl.ds(hi - LANES, LANES)
            keep = lanes < main_valid
            f_v[s_last] = jnp.where(keep, f_v[s_last], 0)
            val_v[s_last] = jnp.where(keep, val_v[s_last], 0.0)

    def fire_gather(p):
        lo, hi = bounds[p], bounds[p + 1]
        return pltpu.async_copy(tab_hbm.at[f_v.at[pl.ds(lo, hi - lo)]],
                                g_v.at[pl.ds(lo, hi - lo)], gsems[p])

    def run_mac(lo, hi, acc):
        @plsc.parallel_loop(lo, hi, step=LANES, unroll=8, carry=acc)
        def macs(i, a):
            s = pl.ds(i, LANES)
            return a + g_v[s] * val_v[s]
        return macs

    run_idx(bounds[0], bounds[1], False)
    gcps = [fire_gather(0)]
    run_idx(bounds[1], bounds[2], False)
    gcps.append(fire_gather(1))
    run_idx(bounds[2], bounds[3], False)
    cp_v.wait()
    run_idx(bounds[3], bounds[4], True)
    gcps.append(fire_gather(2))
    gcps.append(fire_gather(3))

    acc = jnp.zeros((LANES,), jnp.float32)
    for p in range(nph):
        gcps[p].wait()
        acc = run_mac(bounds[p], bounds[p + 1], acc)
    acc_v[...] = acc

    if rem:
        @pl.when(wid == NW - 1)
        def _tail():
            tbase = NW * chunk
            tp_a = pltpu.async_copy(a_hbm.at[pl.ds(tbase, rem)],
                                    ta_v.at[pl.ds(0, rem)], sem)
            tp_b = pltpu.async_copy(b_hbm.at[pl.ds(tbase, rem)],
                                    tb_v.at[pl.ds(0, rem)], sem)
            tp_v = pltpu.async_copy(v_hbm.at[pl.ds(tbase, rem)],
                                    tval_v.at[pl.ds(0, rem)], sem)
            tp_a.wait()
            tp_b.wait()
            tp_v.wait()

            def tidx_body(i, carry):
                s = pl.ds(i * LANES, LANES)
                a = ta_v[s]
                b = tb_v[s]
                tf_v[s] = (((a >> 3) << 15) + ((b >> 7) << 10)
                           + ((a & 7) << 7) + (b & 127))
                return carry

            lax.fori_loop(0, tsteps, tidx_body, 0)
            ts_last = pl.ds((tsteps - 1) * LANES, LANES)
            tkeep = lanes < tail_valid
            tf_v[ts_last] = jnp.where(tkeep, tf_v[ts_last], 0)
            tval_v[ts_last] = jnp.where(tkeep, tval_v[ts_last], 0.0)

            pltpu.async_copy(tab_hbm.at[tf_v], tg_v, sem).wait()

            def tdot_body(i, acc):
                s = pl.ds(i * LANES, LANES)
                return acc + tg_v[s] * tval_v[s]

            tacc = lax.fori_loop(0, tsteps, tdot_body,
                                 jnp.zeros((LANES,), jnp.float32))
            acc_v[...] = acc_v[...] + tacc

    pltpu.sync_copy(acc_v, out_hbm.at[pl.ds(wid * LANES, LANES)])


def kernel(a_idx, b_idx, values, coefficients):
    n = a_idx.shape[0]
    chunk = (n // (NW * 8)) * 8
    rem = n - NW * chunk
    cbuf = -(-chunk // LANES) * LANES
    tbuf = max(-(-rem // LANES) * LANES, LANES)
    tab = (coefficients.reshape(A // 8, 8, B // 128, 128)
           .transpose(0, 2, 1, 3).reshape(A * B))

    mesh = plsc.VectorSubcoreMesh(core_axis_name="c", subcore_axis_name="s")
    out = pl.kernel(
        functools.partial(_body, n),
        out_type=jax.ShapeDtypeStruct((NW * LANES,), jnp.float32),
        mesh=mesh,
        scratch_types=[
            pltpu.VMEM((cbuf,), jnp.int32),
            pltpu.VMEM((cbuf,), jnp.int32),
            pltpu.VMEM((cbuf,), jnp.int32),
            pltpu.VMEM((cbuf,), jnp.float32),
            pltpu.VMEM((cbuf,), jnp.float32),
            pltpu.VMEM((tbuf,), jnp.int32),
            pltpu.VMEM((tbuf,), jnp.int32),
            pltpu.VMEM((tbuf,), jnp.int32),
            pltpu.VMEM((tbuf,), jnp.float32),
            pltpu.VMEM((tbuf,), jnp.float32),
            pltpu.VMEM((LANES,), jnp.float32),
            pltpu.SemaphoreType.DMA,
            pltpu.SemaphoreType.DMA,
            pltpu.SemaphoreType.DMA,
            pltpu.SemaphoreType.DMA,
            pltpu.SemaphoreType.DMA,
        ],
    )(a_idx, b_idx, values, tab)
    return jnp.sum(out)


# speedup vs baseline: 2.7416x; 1.0078x over previous
import functools

import jax
import jax.numpy as jnp
from jax import lax
from jax.experimental import pallas as pl
from jax.experimental.pallas import tpu as pltpu
from jax.experimental.pallas import tpu_sc as plsc

A = 4096
B = 4096
NC = 2
NS = 16
NW = NC * NS
LANES = 16


def _body(n, a_hbm, b_hbm, v_hbm, tab_hbm, out_hbm,
          a_v, b_v, f_v, val_v, g_v,
          ta_v, tb_v, tf_v, tval_v, tg_v, acc_v, sem,
          gsem1, gsem2, gsem3, gsem4):
    chunk = (n // (NW * 8)) * 8
    rem = n - NW * chunk
    vsteps = -(-chunk // LANES)
    main_valid = chunk - (vsteps - 1) * LANES
    tsteps = -(-rem // LANES)
    tail_valid = rem - (tsteps - 1) * LANES

    cid = lax.axis_index("c")
    sid = lax.axis_index("s")
    wid = sid * NC + cid
    base = wid * chunk

    nph = 4
    p0 = vsteps // 8
    per = ((vsteps - p0) // (nph - 1))
    bounds = [0, p0 * LANES] + [
        (p0 + p * per) * LANES for p in range(1, nph - 1)
    ] + [vsteps * LANES]
    d0 = bounds[1]

    cp_a0 = pltpu.async_copy(a_hbm.at[pl.ds(base, d0)],
                             a_v.at[pl.ds(0, d0)], sem)
    cp_b0 = pltpu.async_copy(b_hbm.at[pl.ds(base, d0)],
                             b_v.at[pl.ds(0, d0)], sem)
    cp_a = pltpu.async_copy(a_hbm.at[pl.ds(base + d0, chunk - d0)],
                            a_v.at[pl.ds(d0, chunk - d0)], sem)
    cp_b = pltpu.async_copy(b_hbm.at[pl.ds(base + d0, chunk - d0)],
                            b_v.at[pl.ds(d0, chunk - d0)], sem)
    cp_v = pltpu.async_copy(v_hbm.at[pl.ds(base, chunk)],
                            val_v.at[pl.ds(0, chunk)], sem)
    cp_a0.wait()
    cp_b0.wait()

    lanes = lax.iota(jnp.int32, LANES)
    gsems = [gsem1, gsem2, gsem3, gsem4]

    def run_idx(lo, hi, mask_tail):
        @plsc.parallel_loop(lo, hi, step=LANES, unroll=8)
        def _(i):
            s = pl.ds(i, LANES)
            a = a_v[s]
            b = b_v[s]
            f_v[s] = (((a >> 3) << 15) + ((b >> 7) << 10)
                      + ((a & 7) << 7) + (b & 127))
        if mask_tail:
            s_last = pl.ds(hi - LANES, LANES)
            keep = lanes < main_valid
            f_v[s_last] = jnp.where(keep, f_v[s_last], 0)
            val_v[s_last] = jnp.where(keep, val_v[s_last], 0.0)

    def fire_gather(p):
        lo, hi = bounds[p], bounds[p + 1]
        return pltpu.async_copy(tab_hbm.at[f_v.at[pl.ds(lo, hi - lo)]],
                                g_v.at[pl.ds(lo, hi - lo)], gsems[p])

    def run_mac(lo, hi, acc):
        @plsc.parallel_loop(lo, hi, step=LANES, unroll=8, carry=acc)
        def macs(i, a):
            s = pl.ds(i, LANES)
            return a + g_v[s] * val_v[s]
        return macs

    run_idx(bounds[0], bounds[1], False)
    gcps = [fire_gather(0)]
    cp_a.wait()
    cp_b.wait()
    run_idx(bounds[1], bounds[2], False)
    gcps.append(fire_gather(1))
    run_idx(bounds[2], bounds[3], False)
    cp_v.wait()
    run_idx(bounds[3], bounds[4], True)
    gcps.append(fire_gather(2))
    gcps.append(fire_gather(3))

    acc = jnp.zeros((LANES,), jnp.float32)
    for p in range(nph):
        gcps[p].wait()
        acc = run_mac(bounds[p], bounds[p + 1], acc)
    acc_v[...] = acc

    if rem:
        @pl.when(wid == NW - 1)
        def _tail():
            tbase = NW * chunk
            tp_a = pltpu.async_copy(a_hbm.at[pl.ds(tbase, rem)],
                                    ta_v.at[pl.ds(0, rem)], sem)
            tp_b = pltpu.async_copy(b_hbm.at[pl.ds(tbase, rem)],
                                    tb_v.at[pl.ds(0, rem)], sem)
            tp_v = pltpu.async_copy(v_hbm.at[pl.ds(tbase, rem)],
                                    tval_v.at[pl.ds(0, rem)], sem)
            tp_a.wait()
            tp_b.wait()
            tp_v.wait()

            def tidx_body(i, carry):
                s = pl.ds(i * LANES, LANES)
                a = ta_v[s]
                b = tb_v[s]
                tf_v[s] = (((a >> 3) << 15) + ((b >> 7) << 10)
                           + ((a & 7) << 7) + (b & 127))
                return carry

            lax.fori_loop(0, tsteps, tidx_body, 0)
            ts_last = pl.ds((tsteps - 1) * LANES, LANES)
            tkeep = lanes < tail_valid
            tf_v[ts_last] = jnp.where(tkeep, tf_v[ts_last], 0)
            tval_v[ts_last] = jnp.where(tkeep, tval_v[ts_last], 0.0)

            pltpu.async_copy(tab_hbm.at[tf_v], tg_v, sem).wait()

            def tdot_body(i, acc):
                s = pl.ds(i * LANES, LANES)
                return acc + tg_v[s] * tval_v[s]

            tacc = lax.fori_loop(0, tsteps, tdot_body,
                                 jnp.zeros((LANES,), jnp.float32))
            acc_v[...] = acc_v[...] + tacc

    pltpu.sync_copy(acc_v, out_hbm.at[pl.ds(wid * LANES, LANES)])


def kernel(a_idx, b_idx, values, coefficients):
    n = a_idx.shape[0]
    chunk = (n // (NW * 8)) * 8
    rem = n - NW * chunk
    cbuf = -(-chunk // LANES) * LANES
    tbuf = max(-(-rem // LANES) * LANES, LANES)
    tab = (coefficients.reshape(A // 8, 8, B // 128, 128)
           .transpose(0, 2, 1, 3).reshape(A * B))

    mesh = plsc.VectorSubcoreMesh(core_axis_name="c", subcore_axis_name="s")
    out = pl.kernel(
        functools.partial(_body, n),
        out_type=jax.ShapeDtypeStruct((NW * LANES,), jnp.float32),
        mesh=mesh,
        scratch_types=[
            pltpu.VMEM((cbuf,), jnp.int32),
            pltpu.VMEM((cbuf,), jnp.int32),
            pltpu.VMEM((cbuf,), jnp.int32),
            pltpu.VMEM((cbuf,), jnp.float32),
            pltpu.VMEM((cbuf,), jnp.float32),
            pltpu.VMEM((tbuf,), jnp.int32),
            pltpu.VMEM((tbuf,), jnp.int32),
            pltpu.VMEM((tbuf,), jnp.int32),
            pltpu.VMEM((tbuf,), jnp.float32),
            pltpu.VMEM((tbuf,), jnp.float32),
            pltpu.VMEM((LANES,), jnp.float32),
            pltpu.SemaphoreType.DMA,
            pltpu.SemaphoreType.DMA,
            pltpu.SemaphoreType.DMA,
            pltpu.SemaphoreType.DMA,
            pltpu.SemaphoreType.DMA,
        ],
    )(a_idx, b_idx, values, tab)
    return jnp.sum(out)
